# baseline (speedup 1.0000x reference)
import functools

import jax
import jax.numpy as jnp
from jax import lax
from jax.experimental import pallas as pl
from jax.experimental.pallas import tpu as pltpu
from jax.experimental.pallas import tpu_sc as plsc

_N = 10000
_E = 320000
_D = 128
_H = 128
_C = 16
_EPS = 1e-05

_NC = 2
_NS = 16
_NW = _NC * _NS
_EPW = _E // _NW
_K = 80
_NCHUNK = _EPW // _K
_RPT = 624
_REM = _N - _RPT * _NS

_mesh = plsc.VectorSubcoreMesh(core_axis_name="c", subcore_axis_name="s")


@functools.partial(
    pl.kernel,
    mesh=_mesh,
    out_type=jax.ShapeDtypeStruct((_NC, _N, _H), jnp.float32),
    scratch_types=[
        pltpu.VMEM((_EPW,), jnp.int32),
        pltpu.VMEM((_NCHUNK, _K), jnp.int32),
        pltpu.VMEM((2, _K), jnp.float32),
        pltpu.VMEM((2, _K, _H), jnp.float32),
        pltpu.VMEM_SHARED((_N, _H), jnp.float32),
        pltpu.SemaphoreType.DMA,
        pltpu.SemaphoreType.DMA,
    ],
)
def _spmm_sc(src_hbm, dst3_hbm, w_hbm, h_hbm, zeros_hbm, out_hbm,
             src_v, dst_v, w_v, rows_v, acc, gsem, ssem):
    cid = lax.axis_index("c")
    sid = lax.axis_index("s")
    wid = sid * _NC + cid

    pltpu.sync_copy(zeros_hbm, acc.at[pl.ds(sid * _RPT, _RPT)])

    @pl.when(sid == _NS - 1)
    def _():
        pltpu.sync_copy(zeros_hbm.at[pl.ds(0, _REM)],
                        acc.at[pl.ds(_RPT * _NS, _REM)])

    pltpu.sync_copy(src_hbm.at[pl.ds(wid * _EPW, _EPW)], src_v)
    pltpu.sync_copy(dst3_hbm.at[wid], dst_v)
    plsc.subcore_barrier()

    def gather_chunk(c, buf):
        return (
            pltpu.make_async_copy(
                h_hbm.at[src_v.at[pl.ds(c * _K, _K)]], rows_v.at[buf], gsem),
            pltpu.make_async_copy(
                w_hbm.at[pl.ds(wid * _EPW + c * _K, _K)], w_v.at[buf], gsem),
        )

    def scatter_chunk(c, buf):
        return pltpu.make_async_copy(
            rows_v.at[buf], acc.at[dst_v.at[c]], ssem)

    for cp in gather_chunk(0, 0):
        cp.start()

    def chunk_body(c, carry):
        b = lax.rem(c, 2)
        for cp in gather_chunk(c, b):
            cp.wait()

        @pl.when(c + 1 < _NCHUNK)
        def _():
            for cp in gather_chunk(c + 1, 1 - b):
                cp.start()

        rv = rows_v.at[b]
        wv = w_v.at[b]

        def scale_body(g, c2):
            w16 = wv[pl.ds(g * 16, 16)]
            for i in range(16):
                ws = w16[i]
                e = g * 16 + i
                for col in range(_H // 16):
                    sl = pl.ds(col * 16, 16)
                    rv[e, sl] = rv[e, sl] * ws
            return c2

        lax.fori_loop(0, _K // 16, scale_body, 0)
        return carry

    lax.fori_loop(0, _NCHUNK, chunk_body, 0)
    plsc.subcore_barrier()
    pltpu.sync_copy(acc.at[pl.ds(sid * _RPT, _RPT)],
                    out_hbm.at[cid, pl.ds(sid * _RPT, _RPT)])

    @pl.when(sid == _NS - 1)
    def _():
        pltpu.sync_copy(acc.at[pl.ds(_RPT * _NS, _REM)],
                        out_hbm.at[cid, pl.ds(_RPT * _NS, _REM)])


_GPW = 2 * 2048 // _NW


@functools.partial(
    pl.kernel,
    mesh=_mesh,
    out_type=jax.ShapeDtypeStruct((2 * 2048, _H), jnp.float32),
    scratch_types=[
        pltpu.VMEM((_GPW,), jnp.int32),
        pltpu.VMEM((_GPW, _H), jnp.float32),
        pltpu.SemaphoreType.DMA,
    ],
)
def _gather_sc(table_hbm, idx_hbm, out_hbm, idx_v, rows_v, sem):
    cid = lax.axis_index("c")
    sid = lax.axis_index("s")
    wid = sid * _NC + cid
    base = wid * _GPW
    pltpu.sync_copy(idx_hbm.at[pl.ds(base, _GPW)], idx_v)
    pltpu.async_copy(table_hbm.at[idx_v], rows_v, sem).wait()
    pltpu.sync_copy(rows_v, out_hbm.at[pl.ds(base, _GPW)])


_BLK = 1000


def _mm0_body(x_ref, w_ref, b_ref, o_ref):
    o_ref[...] = (
        jnp.dot(x_ref[...], w_ref[...], preferred_element_type=jnp.float32)
        + b_ref[...]
    )


def _mm0(x, W, b2):
    return pl.pallas_call(
        _mm0_body,
        grid=(_N // _BLK,),
        in_specs=[
            pl.BlockSpec((_BLK, _D), lambda i: (i, 0)),
            pl.BlockSpec((_D, _H), lambda i: (0, 0)),
            pl.BlockSpec((1, _H), lambda i: (0, 0)),
        ],
        out_specs=pl.BlockSpec((_BLK, _H), lambda i: (i, 0)),
        out_shape=jax.ShapeDtypeStruct((_N, _H), jnp.float32),
    )(x, W, b2)


def _fuse_body(p_ref, s_ref, t_ref, w_ref, o_ref):
    x = p_ref[0] + p_ref[1]
    y = jnp.maximum(x * s_ref[...] + t_ref[...], 0.0)
    o_ref[...] = jnp.dot(y, w_ref[...], preferred_element_type=jnp.float32)


def _fuse_mm(p, scale2, shift2, W):
    out_cols = W.shape[1]
    return pl.pallas_call(
        _fuse_body,
        grid=(_N // _BLK,),
        in_specs=[
            pl.BlockSpec((_NC, _BLK, _H), lambda i: (0, i, 0)),
            pl.BlockSpec((1, _H), lambda i: (0, 0)),
            pl.BlockSpec((1, _H), lambda i: (0, 0)),
            pl.BlockSpec((_H, out_cols), lambda i: (0, 0)),
        ],
        out_specs=pl.BlockSpec((_BLK, out_cols), lambda i: (i, 0)),
        out_shape=jax.ShapeDtypeStruct((_N, out_cols), jnp.float32),
    )(p, scale2, shift2, W)


def _fuse_mm_bias_body(p_ref, s_ref, t_ref, w_ref, b_ref, o_ref):
    x = p_ref[0] + p_ref[1]
    y = jnp.maximum(x * s_ref[...] + t_ref[...], 0.0)
    o_ref[...] = (
        jnp.dot(y, w_ref[...], preferred_element_type=jnp.float32) + b_ref[...]
    )


def _fuse_mm_bias(p, scale2, shift2, W, b2, blk):
    rows = p.shape[1]
    out_cols = W.shape[1]
    return pl.pallas_call(
        _fuse_mm_bias_body,
        grid=(rows // blk,),
        in_specs=[
            pl.BlockSpec((_NC, blk, _H), lambda i: (0, i, 0)),
            pl.BlockSpec((1, _H), lambda i: (0, 0)),
            pl.BlockSpec((1, _H), lambda i: (0, 0)),
            pl.BlockSpec((_H, out_cols), lambda i: (0, 0)),
            pl.BlockSpec((1, out_cols), lambda i: (0, 0)),
        ],
        out_specs=pl.BlockSpec((blk, out_cols), lambda i: (i, 0)),
        out_shape=jax.ShapeDtypeStruct((rows, out_cols), jnp.float32),
    )(p, scale2, shift2, W, b2)


def kernel(features, edge_index, edge_weight, idx,
           W0, b0, bias0, gamma0, beta0, mean0, var0,
           W1, bias1, gamma1, beta1, mean1, var1,
           Wf, bf):
    src = edge_index[0]
    dst = edge_index[1].reshape(_NW, _NCHUNK, _K)
    zeros = jnp.zeros((_RPT, _H), jnp.float32)

    scale0 = lax.rsqrt(var0 + _EPS) * gamma0
    shift0 = beta0 + (bias0 - mean0) * scale0
    scale1 = lax.rsqrt(var1 + _EPS) * gamma1
    shift1 = beta1 + (bias1 - mean1) * scale1

    h = _mm0(features, W0, b0.reshape(1, _H))
    p = _spmm_sc(src, dst, edge_weight, h, zeros)
    h = _fuse_mm(p, scale0.reshape(1, _H), shift0.reshape(1, _H), W1)
    p = _spmm_sc(src, dst, edge_weight, h, zeros)
    idx2 = jnp.concatenate([idx, idx + _N])
    g = _gather_sc(p.reshape(2 * _N, _H), idx2)
    g = g.reshape(_NC, 2048, _H)
    return _fuse_mm_bias(g, scale1.reshape(1, _H), shift1.reshape(1, _H),
                         Wf, bf.reshape(1, _C), 1024)


# speedup vs baseline: 2.1271x; 2.1271x over previous
import functools

import jax
import jax.numpy as jnp
from jax import lax
from jax.experimental import pallas as pl
from jax.experimental.pallas import tpu as pltpu
from jax.experimental.pallas import tpu_sc as plsc

_N = 10000
_E = 320000
_D = 128
_H = 128
_C = 16
_EPS = 1e-05

_NC = 2
_NS = 16
_NW = _NC * _NS
_EPW = _E // _NW
_K = 80
_NCHUNK = _EPW // _K
_RPT = 624
_REM = _N - _RPT * _NS

_mesh = plsc.VectorSubcoreMesh(core_axis_name="c", subcore_axis_name="s")


@functools.partial(
    pl.kernel,
    mesh=_mesh,
    out_type=jax.ShapeDtypeStruct((_NC, _N, _H), jnp.float32),
    scratch_types=[
        pltpu.VMEM((_EPW,), jnp.int32),
        pltpu.VMEM((_NCHUNK, _K), jnp.int32),
        pltpu.VMEM((2, _K), jnp.float32),
        pltpu.VMEM((2, _K, _H), jnp.float32),
        pltpu.VMEM_SHARED((_N, _H), jnp.float32),
        pltpu.SemaphoreType.DMA,
        pltpu.SemaphoreType.DMA,
    ],
)
def _spmm_sc(src_hbm, dst3_hbm, w_hbm, h_hbm, zeros_hbm, out_hbm,
             src_v, dst_v, w_v, rows_v, acc, gsem, ssem):
    cid = lax.axis_index("c")
    sid = lax.axis_index("s")
    wid = sid * _NC + cid

    pltpu.sync_copy(zeros_hbm, acc.at[pl.ds(sid * _RPT, _RPT)])

    @pl.when(sid == _NS - 1)
    def _():
        pltpu.sync_copy(zeros_hbm.at[pl.ds(0, _REM)],
                        acc.at[pl.ds(_RPT * _NS, _REM)])

    pltpu.sync_copy(src_hbm.at[pl.ds(wid * _EPW, _EPW)], src_v)
    pltpu.sync_copy(dst3_hbm.at[wid], dst_v)
    plsc.subcore_barrier()

    def gather_chunk(c, buf):
        return (
            pltpu.make_async_copy(
                h_hbm.at[src_v.at[pl.ds(c * _K, _K)]], rows_v.at[buf], gsem),
            pltpu.make_async_copy(
                w_hbm.at[pl.ds(wid * _EPW + c * _K, _K)], w_v.at[buf], gsem),
        )

    def scatter_chunk(c, buf):
        return pltpu.make_async_copy(
            rows_v.at[buf], acc.at[dst_v.at[c]], ssem)

    def scale_chunk(b):
        rv = rows_v.at[b]
        wv = w_v.at[b]
        for g in range(_K // 16):
            w16 = wv[pl.ds(g * 16, 16)]
            for i in range(16):
                ws = w16[i]
                e = g * 16 + i
                for col in range(_H // 16):
                    sl = pl.ds(col * 16, 16)
                    rv[e, sl] = rv[e, sl] * ws

    def chunk_body(c, b, first, guard_prefetch):
        for cp in gather_chunk(c, b):
            cp.wait()
        if not first:
            scatter_chunk(c - 1, 1 - b).wait()
        if guard_prefetch:
            @pl.when(c + 1 < _NCHUNK)
            def _():
                for cp in gather_chunk(c + 1, 1 - b):
                    cp.start()
        else:
            for cp in gather_chunk(c + 1, 1 - b):
                cp.start()
        scale_chunk(b)
        scatter_chunk(c, b).start(add=True)

    for cp in gather_chunk(0, 0):
        cp.start()
    chunk_body(0, 0, True, False)

    def outer_body(t, carry):
        chunk_body(2 * t + 1, 1, False, False)
        chunk_body(2 * t + 2, 0, False, True)
        return carry

    lax.fori_loop(0, (_NCHUNK - 1) // 2, outer_body, 0)
    scatter_chunk(_NCHUNK - 1, 0).wait()
    plsc.subcore_barrier()
    pltpu.sync_copy(acc.at[pl.ds(sid * _RPT, _RPT)],
                    out_hbm.at[cid, pl.ds(sid * _RPT, _RPT)])

    @pl.when(sid == _NS - 1)
    def _():
        pltpu.sync_copy(acc.at[pl.ds(_RPT * _NS, _REM)],
                        out_hbm.at[cid, pl.ds(_RPT * _NS, _REM)])


_GPW = 2 * 2048 // _NW


@functools.partial(
    pl.kernel,
    mesh=_mesh,
    out_type=jax.ShapeDtypeStruct((2 * 2048, _H), jnp.float32),
    scratch_types=[
        pltpu.VMEM((_GPW,), jnp.int32),
        pltpu.VMEM((_GPW, _H), jnp.float32),
        pltpu.SemaphoreType.DMA,
    ],
)
def _gather_sc(table_hbm, idx_hbm, out_hbm, idx_v, rows_v, sem):
    cid = lax.axis_index("c")
    sid = lax.axis_index("s")
    wid = sid * _NC + cid
    base = wid * _GPW
    pltpu.sync_copy(idx_hbm.at[pl.ds(base, _GPW)], idx_v)
    pltpu.async_copy(table_hbm.at[idx_v], rows_v, sem).wait()
    pltpu.sync_copy(rows_v, out_hbm.at[pl.ds(base, _GPW)])


_BLK = 1000


def _mm0_body(x_ref, w_ref, b_ref, o_ref):
    o_ref[...] = (
        jnp.dot(x_ref[...], w_ref[...], preferred_element_type=jnp.float32)
        + b_ref[...]
    )


def _mm0(x, W, b2):
    return pl.pallas_call(
        _mm0_body,
        grid=(_N // _BLK,),
        in_specs=[
            pl.BlockSpec((_BLK, _D), lambda i: (i, 0)),
            pl.BlockSpec((_D, _H), lambda i: (0, 0)),
            pl.BlockSpec((1, _H), lambda i: (0, 0)),
        ],
        out_specs=pl.BlockSpec((_BLK, _H), lambda i: (i, 0)),
        out_shape=jax.ShapeDtypeStruct((_N, _H), jnp.float32),
    )(x, W, b2)


def _fuse_body(p_ref, s_ref, t_ref, w_ref, o_ref):
    x = p_ref[0] + p_ref[1]
    y = jnp.maximum(x * s_ref[...] + t_ref[...], 0.0)
    o_ref[...] = jnp.dot(y, w_ref[...], preferred_element_type=jnp.float32)


def _fuse_mm(p, scale2, shift2, W):
    out_cols = W.shape[1]
    return pl.pallas_call(
        _fuse_body,
        grid=(_N // _BLK,),
        in_specs=[
            pl.BlockSpec((_NC, _BLK, _H), lambda i: (0, i, 0)),
            pl.BlockSpec((1, _H), lambda i: (0, 0)),
            pl.BlockSpec((1, _H), lambda i: (0, 0)),
            pl.BlockSpec((_H, out_cols), lambda i: (0, 0)),
        ],
        out_specs=pl.BlockSpec((_BLK, out_cols), lambda i: (i, 0)),
        out_shape=jax.ShapeDtypeStruct((_N, out_cols), jnp.float32),
    )(p, scale2, shift2, W)


def _fuse_mm_bias_body(p_ref, s_ref, t_ref, w_ref, b_ref, o_ref):
    x = p_ref[0] + p_ref[1]
    y = jnp.maximum(x * s_ref[...] + t_ref[...], 0.0)
    o_ref[...] = (
        jnp.dot(y, w_ref[...], preferred_element_type=jnp.float32) + b_ref[...]
    )


def _fuse_mm_bias(p, scale2, shift2, W, b2, blk):
    rows = p.shape[1]
    out_cols = W.shape[1]
    return pl.pallas_call(
        _fuse_mm_bias_body,
        grid=(rows // blk,),
        in_specs=[
            pl.BlockSpec((_NC, blk, _H), lambda i: (0, i, 0)),
            pl.BlockSpec((1, _H), lambda i: (0, 0)),
            pl.BlockSpec((1, _H), lambda i: (0, 0)),
            pl.BlockSpec((_H, out_cols), lambda i: (0, 0)),
            pl.BlockSpec((1, out_cols), lambda i: (0, 0)),
        ],
        out_specs=pl.BlockSpec((blk, out_cols), lambda i: (i, 0)),
        out_shape=jax.ShapeDtypeStruct((rows, out_cols), jnp.float32),
    )(p, scale2, shift2, W, b2)


def kernel(features, edge_index, edge_weight, idx,
           W0, b0, bias0, gamma0, beta0, mean0, var0,
           W1, bias1, gamma1, beta1, mean1, var1,
           Wf, bf):
    src = edge_index[0]
    dst = edge_index[1].reshape(_NW, _NCHUNK, _K)
    zeros = jnp.zeros((_RPT, _H), jnp.float32)

    scale0 = lax.rsqrt(var0 + _EPS) * gamma0
    shift0 = beta0 + (bias0 - mean0) * scale0
    scale1 = lax.rsqrt(var1 + _EPS) * gamma1
    shift1 = beta1 + (bias1 - mean1) * scale1

    h = _mm0(features, W0, b0.reshape(1, _H))
    p = _spmm_sc(src, dst, edge_weight, h, zeros)
    h = _fuse_mm(p, scale0.reshape(1, _H), shift0.reshape(1, _H), W1)
    p = _spmm_sc(src, dst, edge_weight, h, zeros)
    idx2 = jnp.concatenate([idx, idx + _N])
    g = _gather_sc(p.reshape(2 * _N, _H), idx2)
    g = g.reshape(_NC, 2048, _H)
    return _fuse_mm_bias(g, scale1.reshape(1, _H), shift1.reshape(1, _H),
                         Wf, bf.reshape(1, _C), 1024)


# speedup vs baseline: 2.4771x; 1.1645x over previous
import functools

import jax
import jax.numpy as jnp
from jax import lax
from jax.experimental import pallas as pl
from jax.experimental.pallas import tpu as pltpu
from jax.experimental.pallas import tpu_sc as plsc

_N = 10000
_E = 320000
_D = 128
_H = 128
_C = 16
_EPS = 1e-05

_NC = 2
_NS = 16
_NW = _NC * _NS
_EPW = _E // _NW
_K = 80
_NCHUNK = _EPW // _K
_RPT = 624
_REM = _N - _RPT * _NS

_mesh = plsc.VectorSubcoreMesh(core_axis_name="c", subcore_axis_name="s")


@functools.partial(
    pl.kernel,
    mesh=_mesh,
    out_type=jax.ShapeDtypeStruct((_NC, _N, _H), jnp.float32),
    scratch_types=[
        pltpu.VMEM((_EPW,), jnp.int32),
        pltpu.VMEM((2, _K), jnp.int32),
        pltpu.VMEM((2, _K), jnp.float32),
        pltpu.VMEM((2, _K, _H), jnp.bfloat16),
        pltpu.VMEM((2, _K, _H), jnp.float32),
        pltpu.VMEM_SHARED((_N, _H), jnp.float32),
        pltpu.SemaphoreType.DMA,
        pltpu.SemaphoreType.DMA,
    ],
    compiler_params=pltpu.CompilerParams(
        use_tc_tiling_on_sc=False, needs_layout_passes=False),
)
def _spmm_sc(src_hbm, dst3_hbm, w_hbm, h_hbm, zeros_hbm, out_hbm,
             src_v, dst_v, w_v, rows_v, sbuf_v, acc, gsem, ssem):
    cid = lax.axis_index("c")
    sid = lax.axis_index("s")
    wid = sid * _NC + cid

    pltpu.sync_copy(zeros_hbm, acc.at[pl.ds(sid * _RPT, _RPT)])

    @pl.when(sid == _NS - 1)
    def _():
        pltpu.sync_copy(zeros_hbm.at[pl.ds(0, _REM)],
                        acc.at[pl.ds(_RPT * _NS, _REM)])

    pltpu.sync_copy(src_hbm.at[pl.ds(wid * _EPW, _EPW)], src_v)
    plsc.subcore_barrier()

    def gather_chunk(c, buf):
        return (
            pltpu.make_async_copy(
                h_hbm.at[src_v.at[pl.ds(c * _K, _K)]], rows_v.at[buf], gsem),
            pltpu.make_async_copy(
                w_hbm.at[pl.ds(wid * _EPW + c * _K, _K)], w_v.at[buf], gsem),
            pltpu.make_async_copy(
                dst3_hbm.at[wid, c], dst_v.at[buf], gsem),
        )

    def scatter_chunk(c, buf):
        return pltpu.make_async_copy(
            sbuf_v.at[buf], acc.at[dst_v.at[buf]], ssem)

    def scale_chunk(b):
        rv = rows_v.at[b]
        sb = sbuf_v.at[b]
        wv = w_v.at[b]
        for g in range(_K // 16):
            w16 = wv[pl.ds(g * 16, 16)]
            for i in range(16):
                ws = w16[i]
                e = g * 16 + i
                for q in range(_H // 32):
                    bb = rv[e, pl.ds(32 * q, 32)]
                    lo, hi = plsc.unpack(bb, format=plsc.PackFormat.INTERLEAVED)
                    sb[e, pl.ds(32 * q, 16)] = lo * ws
                    sb[e, pl.ds(32 * q + 16, 16)] = hi * ws

    def chunk_body(c, b, first, guard_prefetch):
        for cp in gather_chunk(c, b):
            cp.wait()
        if not first:
            scatter_chunk(c - 1, 1 - b).wait()
        if guard_prefetch:
            @pl.when(c + 1 < _NCHUNK)
            def _():
                for cp in gather_chunk(c + 1, 1 - b):
                    cp.start()
        else:
            for cp in gather_chunk(c + 1, 1 - b):
                cp.start()
        scale_chunk(b)
        scatter_chunk(c, b).start(add=True)

    for cp in gather_chunk(0, 0):
        cp.start()
    chunk_body(0, 0, True, False)

    def outer_body(t, carry):
        chunk_body(2 * t + 1, 1, False, False)
        chunk_body(2 * t + 2, 0, False, True)
        return carry

    lax.fori_loop(0, (_NCHUNK - 1) // 2, outer_body, 0)
    scatter_chunk(_NCHUNK - 1, 0).wait()
    plsc.subcore_barrier()
    pltpu.sync_copy(acc.at[pl.ds(sid * _RPT, _RPT)],
                    out_hbm.at[cid, pl.ds(sid * _RPT, _RPT)])

    @pl.when(sid == _NS - 1)
    def _():
        pltpu.sync_copy(acc.at[pl.ds(_RPT * _NS, _REM)],
                        out_hbm.at[cid, pl.ds(_RPT * _NS, _REM)])


_GPW = 2 * 2048 // _NW


@functools.partial(
    pl.kernel,
    mesh=_mesh,
    out_type=jax.ShapeDtypeStruct((2 * 2048, _H), jnp.float32),
    scratch_types=[
        pltpu.VMEM((_GPW,), jnp.int32),
        pltpu.VMEM((_GPW, _H), jnp.float32),
        pltpu.SemaphoreType.DMA,
    ],
)
def _gather_sc(table_hbm, idx_hbm, out_hbm, idx_v, rows_v, sem):
    cid = lax.axis_index("c")
    sid = lax.axis_index("s")
    wid = sid * _NC + cid
    base = wid * _GPW
    pltpu.sync_copy(idx_hbm.at[pl.ds(base, _GPW)], idx_v)
    pltpu.async_copy(table_hbm.at[idx_v], rows_v, sem).wait()
    pltpu.sync_copy(rows_v, out_hbm.at[pl.ds(base, _GPW)])


_BLK = 1000


def _mm0_body(x_ref, w_ref, b_ref, o_ref):
    o_ref[...] = (
        jnp.dot(x_ref[...], w_ref[...], preferred_element_type=jnp.float32)
        + b_ref[...]
    )


def _mm0(x, W, b2):
    return pl.pallas_call(
        _mm0_body,
        grid=(_N // _BLK,),
        in_specs=[
            pl.BlockSpec((_BLK, _D), lambda i: (i, 0)),
            pl.BlockSpec((_D, _H), lambda i: (0, 0)),
            pl.BlockSpec((1, _H), lambda i: (0, 0)),
        ],
        out_specs=pl.BlockSpec((_BLK, _H), lambda i: (i, 0)),
        out_shape=jax.ShapeDtypeStruct((_N, _H), jnp.float32),
    )(x, W, b2)


def _fuse_body(p_ref, s_ref, t_ref, w_ref, o_ref):
    x = p_ref[0] + p_ref[1]
    y = jnp.maximum(x * s_ref[...] + t_ref[...], 0.0)
    o_ref[...] = jnp.dot(y, w_ref[...], preferred_element_type=jnp.float32)


def _fuse_mm(p, scale2, shift2, W):
    out_cols = W.shape[1]
    return pl.pallas_call(
        _fuse_body,
        grid=(_N // _BLK,),
        in_specs=[
            pl.BlockSpec((_NC, _BLK, _H), lambda i: (0, i, 0)),
            pl.BlockSpec((1, _H), lambda i: (0, 0)),
            pl.BlockSpec((1, _H), lambda i: (0, 0)),
            pl.BlockSpec((_H, out_cols), lambda i: (0, 0)),
        ],
        out_specs=pl.BlockSpec((_BLK, out_cols), lambda i: (i, 0)),
        out_shape=jax.ShapeDtypeStruct((_N, out_cols), jnp.float32),
    )(p, scale2, shift2, W)


def _fuse_mm_bias_body(p_ref, s_ref, t_ref, w_ref, b_ref, o_ref):
    x = p_ref[0] + p_ref[1]
    y = jnp.maximum(x * s_ref[...] + t_ref[...], 0.0)
    o_ref[...] = (
        jnp.dot(y, w_ref[...], preferred_element_type=jnp.float32) + b_ref[...]
    )


def _fuse_mm_bias(p, scale2, shift2, W, b2, blk):
    rows = p.shape[1]
    out_cols = W.shape[1]
    return pl.pallas_call(
        _fuse_mm_bias_body,
        grid=(rows // blk,),
        in_specs=[
            pl.BlockSpec((_NC, blk, _H), lambda i: (0, i, 0)),
            pl.BlockSpec((1, _H), lambda i: (0, 0)),
            pl.BlockSpec((1, _H), lambda i: (0, 0)),
            pl.BlockSpec((_H, out_cols), lambda i: (0, 0)),
            pl.BlockSpec((1, out_cols), lambda i: (0, 0)),
        ],
        out_specs=pl.BlockSpec((blk, out_cols), lambda i: (i, 0)),
        out_shape=jax.ShapeDtypeStruct((rows, out_cols), jnp.float32),
    )(p, scale2, shift2, W, b2)


_PERM_LIST = [0] * _H
for _j in range(_H // 32):
    for _i in range(16):
        _PERM_LIST[32 * _j + 2 * _i] = 32 * _j + _i
        _PERM_LIST[32 * _j + 2 * _i + 1] = 32 * _j + 16 + _i


def _pack_bf16(h):
    return h.astype(jnp.bfloat16)


def kernel(features, edge_index, edge_weight, idx,
           W0, b0, bias0, gamma0, beta0, mean0, var0,
           W1, bias1, gamma1, beta1, mean1, var1,
           Wf, bf):
    src = edge_index[0]
    dst = edge_index[1].reshape(_NW, _NCHUNK, _K)
    zeros = jnp.zeros((_RPT, _H), jnp.float32)
    perm = jnp.array(_PERM_LIST, dtype=jnp.int32)

    scale0 = lax.rsqrt(var0 + _EPS) * gamma0
    shift0 = beta0 + (bias0 - mean0) * scale0
    scale1 = lax.rsqrt(var1 + _EPS) * gamma1
    shift1 = beta1 + (bias1 - mean1) * scale1

    h = _mm0(features, W0[:, perm], b0[perm].reshape(1, _H))
    p = _spmm_sc(src, dst, edge_weight, _pack_bf16(h), zeros)
    h = _fuse_mm(p, scale0.reshape(1, _H), shift0.reshape(1, _H), W1[:, perm])
    p = _spmm_sc(src, dst, edge_weight, _pack_bf16(h), zeros)
    idx2 = jnp.concatenate([idx, idx + _N])
    g = _gather_sc(p.reshape(2 * _N, _H), idx2)
    g = g.reshape(_NC, 2048, _H)
    return _fuse_mm_bias(g, scale1.reshape(1, _H), shift1.reshape(1, _H),
                         Wf, bf.reshape(1, _C), 1024)


# speedup vs baseline: 2.4898x; 1.0051x over previous
import functools

import jax
import jax.numpy as jnp
from jax import lax
from jax.experimental import pallas as pl
from jax.experimental.pallas import tpu as pltpu
from jax.experimental.pallas import tpu_sc as plsc

_N = 10000
_E = 320000
_D = 128
_H = 128
_C = 16
_EPS = 1e-05

_NC = 2
_NS = 16
_NW = _NC * _NS
_EPW = _E // _NW
_K = 80
_NCHUNK = _EPW // _K
_RPT = 624
_REM = _N - _RPT * _NS

_mesh = plsc.VectorSubcoreMesh(core_axis_name="c", subcore_axis_name="s")


@functools.partial(
    pl.kernel,
    mesh=_mesh,
    out_type=jax.ShapeDtypeStruct((_NC, _N, _H), jnp.float32),
    scratch_types=[
        pltpu.VMEM((_EPW,), jnp.int32),
        pltpu.VMEM((2, _K), jnp.int32),
        pltpu.VMEM((2, _K), jnp.float32),
        pltpu.VMEM((2, _K, _H), jnp.bfloat16),
        pltpu.VMEM((2, _K, _H), jnp.float32),
        pltpu.VMEM_SHARED((_N, _H), jnp.float32),
        pltpu.SemaphoreType.DMA,
        pltpu.SemaphoreType.DMA,
    ],
    compiler_params=pltpu.CompilerParams(
        use_tc_tiling_on_sc=False, needs_layout_passes=False),
)
def _spmm_sc(src_hbm, dst3_hbm, w_hbm, h_hbm, zeros_hbm, out_hbm,
             src_v, dst_v, w_v, rows_v, sbuf_v, acc, gsem, ssem):
    cid = lax.axis_index("c")
    sid = lax.axis_index("s")
    wid = sid * _NC + cid

    pltpu.sync_copy(zeros_hbm, acc.at[pl.ds(sid * _RPT, _RPT)])

    @pl.when(sid == _NS - 1)
    def _():
        pltpu.sync_copy(zeros_hbm.at[pl.ds(0, _REM)],
                        acc.at[pl.ds(_RPT * _NS, _REM)])

    pltpu.sync_copy(src_hbm.at[pl.ds(wid * _EPW, _EPW)], src_v)
    plsc.subcore_barrier()

    def gather_chunk(c, buf):
        return (
            pltpu.make_async_copy(
                h_hbm.at[src_v.at[pl.ds(c * _K, _K)]], rows_v.at[buf], gsem),
            pltpu.make_async_copy(
                w_hbm.at[pl.ds(wid * _EPW + c * _K, _K)], w_v.at[buf], gsem),
            pltpu.make_async_copy(
                dst3_hbm.at[wid, c], dst_v.at[buf], gsem),
        )

    def scatter_chunk(c, buf):
        return pltpu.make_async_copy(
            sbuf_v.at[buf], acc.at[dst_v.at[buf]], ssem)

    def scale_chunk(b):
        rv = rows_v.at[b]
        sb = sbuf_v.at[b]
        wv = w_v.at[b]
        for g in range(_K // 16):
            w16 = wv[pl.ds(g * 16, 16)]
            for i in range(16):
                ws = w16[i]
                e = g * 16 + i
                for q in range(_H // 32):
                    bb = rv[e, pl.ds(32 * q, 32)]
                    lo, hi = plsc.unpack(bb, format=plsc.PackFormat.INTERLEAVED)
                    sb[e, pl.ds(32 * q, 16)] = lo * ws
                    sb[e, pl.ds(32 * q + 16, 16)] = hi * ws

    def chunk_body(c, b, first, guard_prefetch):
        for cp in gather_chunk(c, b):
            cp.wait()
        if not first:
            scatter_chunk(c - 1, 1 - b).wait()
        if guard_prefetch:
            @pl.when(c + 1 < _NCHUNK)
            def _():
                for cp in gather_chunk(c + 1, 1 - b):
                    cp.start()
        else:
            for cp in gather_chunk(c + 1, 1 - b):
                cp.start()
        scale_chunk(b)
        scatter_chunk(c, b).start(add=True)

    for cp in gather_chunk(0, 0):
        cp.start()
    chunk_body(0, 0, True, False)

    def outer_body(t, carry):
        chunk_body(2 * t + 1, 1, False, False)
        chunk_body(2 * t + 2, 0, False, True)
        return carry

    lax.fori_loop(0, (_NCHUNK - 1) // 2, outer_body, 0)
    scatter_chunk(_NCHUNK - 1, 0).wait()
    plsc.subcore_barrier()
    pltpu.sync_copy(acc.at[pl.ds(sid * _RPT, _RPT)],
                    out_hbm.at[cid, pl.ds(sid * _RPT, _RPT)])

    @pl.when(sid == _NS - 1)
    def _():
        pltpu.sync_copy(acc.at[pl.ds(_RPT * _NS, _REM)],
                        out_hbm.at[cid, pl.ds(_RPT * _NS, _REM)])


_GPW = 2 * 2048 // _NW


@functools.partial(
    pl.kernel,
    mesh=_mesh,
    out_type=jax.ShapeDtypeStruct((2 * 2048, _H), jnp.float32),
    scratch_types=[
        pltpu.VMEM((_GPW,), jnp.int32),
        pltpu.VMEM((_GPW, _H), jnp.float32),
        pltpu.SemaphoreType.DMA,
    ],
    compiler_params=pltpu.CompilerParams(
        use_tc_tiling_on_sc=False, needs_layout_passes=False),
)
def _gather_sc(table_hbm, idx_hbm, out_hbm, idx_v, rows_v, sem):
    cid = lax.axis_index("c")
    sid = lax.axis_index("s")
    wid = sid * _NC + cid
    base = wid * _GPW
    pltpu.sync_copy(idx_hbm.at[pl.ds(base, _GPW)], idx_v)
    pltpu.async_copy(table_hbm.at[idx_v], rows_v, sem).wait()
    pltpu.sync_copy(rows_v, out_hbm.at[pl.ds(base, _GPW)])


_BLK = 1000


def _mm0_body(x_ref, w_ref, b_ref, o_ref):
    o_ref[...] = (
        jnp.dot(x_ref[...], w_ref[...], preferred_element_type=jnp.float32)
        + b_ref[...]
    ).astype(jnp.bfloat16)


def _mm0(x, W, b2):
    return pl.pallas_call(
        _mm0_body,
        grid=(_N // _BLK,),
        in_specs=[
            pl.BlockSpec((_BLK, _D), lambda i: (i, 0)),
            pl.BlockSpec((_D, _H), lambda i: (0, 0)),
            pl.BlockSpec((1, _H), lambda i: (0, 0)),
        ],
        out_specs=pl.BlockSpec((_BLK, _H), lambda i: (i, 0)),
        out_shape=jax.ShapeDtypeStruct((_N, _H), jnp.bfloat16),
    )(x, W, b2)


def _fuse_body(p_ref, s_ref, t_ref, w_ref, o_ref):
    x = p_ref[0] + p_ref[1]
    y = jnp.maximum(x * s_ref[...] + t_ref[...], 0.0)
    o_ref[...] = jnp.dot(
        y, w_ref[...], preferred_element_type=jnp.float32
    ).astype(jnp.bfloat16)


def _fuse_mm(p, scale2, shift2, W):
    out_cols = W.shape[1]
    return pl.pallas_call(
        _fuse_body,
        grid=(_N // _BLK,),
        in_specs=[
            pl.BlockSpec((_NC, _BLK, _H), lambda i: (0, i, 0)),
            pl.BlockSpec((1, _H), lambda i: (0, 0)),
            pl.BlockSpec((1, _H), lambda i: (0, 0)),
            pl.BlockSpec((_H, out_cols), lambda i: (0, 0)),
        ],
        out_specs=pl.BlockSpec((_BLK, out_cols), lambda i: (i, 0)),
        out_shape=jax.ShapeDtypeStruct((_N, out_cols), jnp.bfloat16),
    )(p, scale2, shift2, W)


def _fuse_mm_bias_body(p_ref, s_ref, t_ref, w_ref, b_ref, o_ref):
    x = p_ref[0] + p_ref[1]
    y = jnp.maximum(x * s_ref[...] + t_ref[...], 0.0)
    o_ref[...] = (
        jnp.dot(y, w_ref[...], preferred_element_type=jnp.float32) + b_ref[...]
    )


def _fuse_mm_bias(p, scale2, shift2, W, b2, blk):
    rows = p.shape[1]
    out_cols = W.shape[1]
    return pl.pallas_call(
        _fuse_mm_bias_body,
        grid=(rows // blk,),
        in_specs=[
            pl.BlockSpec((_NC, blk, _H), lambda i: (0, i, 0)),
            pl.BlockSpec((1, _H), lambda i: (0, 0)),
            pl.BlockSpec((1, _H), lambda i: (0, 0)),
            pl.BlockSpec((_H, out_cols), lambda i: (0, 0)),
            pl.BlockSpec((1, out_cols), lambda i: (0, 0)),
        ],
        out_specs=pl.BlockSpec((blk, out_cols), lambda i: (i, 0)),
        out_shape=jax.ShapeDtypeStruct((rows, out_cols), jnp.float32),
    )(p, scale2, shift2, W, b2)


_PERM_LIST = [0] * _H
for _j in range(_H // 32):
    for _i in range(16):
        _PERM_LIST[32 * _j + 2 * _i] = 32 * _j + _i
        _PERM_LIST[32 * _j + 2 * _i + 1] = 32 * _j + 16 + _i


def kernel(features, edge_index, edge_weight, idx,
           W0, b0, bias0, gamma0, beta0, mean0, var0,
           W1, bias1, gamma1, beta1, mean1, var1,
           Wf, bf):
    src = edge_index[0]
    dst = edge_index[1].reshape(_NW, _NCHUNK, _K)
    zeros = jnp.zeros((_RPT, _H), jnp.float32)
    perm = jnp.array(_PERM_LIST, dtype=jnp.int32)

    scale0 = lax.rsqrt(var0 + _EPS) * gamma0
    shift0 = beta0 + (bias0 - mean0) * scale0
    scale1 = lax.rsqrt(var1 + _EPS) * gamma1
    shift1 = beta1 + (bias1 - mean1) * scale1

    h = _mm0(features, W0[:, perm], b0[perm].reshape(1, _H))
    p = _spmm_sc(src, dst, edge_weight, h, zeros)
    h = _fuse_mm(p, scale0.reshape(1, _H), shift0.reshape(1, _H), W1[:, perm])
    p = _spmm_sc(src, dst, edge_weight, h, zeros)
    idx2 = jnp.concatenate([idx, idx + _N])
    g = _gather_sc(p.reshape(2 * _N, _H), idx2)
    g = g.reshape(_NC, 2048, _H)
    return _fuse_mm_bias(g, scale1.reshape(1, _H), shift1.reshape(1, _H),
                         Wf, bf.reshape(1, _C), 1024)


# speedup vs baseline: 2.5078x; 1.0072x over previous
import functools

import jax
import jax.numpy as jnp
from jax import lax
from jax.experimental import pallas as pl
from jax.experimental.pallas import tpu as pltpu
from jax.experimental.pallas import tpu_sc as plsc

_N = 10000
_E = 320000
_D = 128
_H = 128
_C = 16
_EPS = 1e-05

_NC = 2
_NS = 16
_NW = _NC * _NS
_EPW = _E // _NW
_K = 80
_NCHUNK = _EPW // _K
_RPT = 624
_REM = _N - _RPT * _NS

_mesh = plsc.VectorSubcoreMesh(core_axis_name="c", subcore_axis_name="s")


@functools.partial(
    pl.kernel,
    mesh=_mesh,
    out_type=jax.ShapeDtypeStruct((_NC, _N, _H), jnp.float32),
    scratch_types=[
        pltpu.VMEM((_EPW,), jnp.int32),
        pltpu.VMEM((3, _K), jnp.int32),
        pltpu.VMEM((2, _K), jnp.float32),
        pltpu.VMEM((2, _K, _H), jnp.bfloat16),
        pltpu.VMEM((2, _K, _H), jnp.float32),
        pltpu.VMEM_SHARED((_N, _H), jnp.float32),
        pltpu.SemaphoreType.DMA,
        pltpu.SemaphoreType.DMA,
    ],
    compiler_params=pltpu.CompilerParams(
        use_tc_tiling_on_sc=False, needs_layout_passes=False),
)
def _spmm_sc(src_hbm, dst3_hbm, w_hbm, h_hbm, zeros_hbm, out_hbm,
             src_v, dst_v, w_v, rows_v, sbuf_v, acc, gsem, ssem):
    cid = lax.axis_index("c")
    sid = lax.axis_index("s")
    wid = sid * _NC + cid

    pltpu.sync_copy(zeros_hbm, acc.at[pl.ds(sid * _RPT, _RPT)])

    @pl.when(sid == _NS - 1)
    def _():
        pltpu.sync_copy(zeros_hbm.at[pl.ds(0, _REM)],
                        acc.at[pl.ds(_RPT * _NS, _REM)])

    pltpu.sync_copy(src_hbm.at[pl.ds(wid * _EPW, _EPW)], src_v)
    plsc.subcore_barrier()

    def gather_chunk(c, buf):
        return (
            pltpu.make_async_copy(
                h_hbm.at[src_v.at[pl.ds(c * _K, _K)]], rows_v.at[buf], gsem),
            pltpu.make_async_copy(
                w_hbm.at[pl.ds(wid * _EPW + c * _K, _K)], w_v.at[buf], gsem),
            pltpu.make_async_copy(
                dst3_hbm.at[wid, c], dst_v.at[lax.rem(c, 3)], gsem),
        )

    def scatter_chunk(c, buf):
        return pltpu.make_async_copy(
            sbuf_v.at[buf], acc.at[dst_v.at[lax.rem(c, 3)]], ssem)

    def scale_chunk(b):
        rv = rows_v.at[b]
        sb = sbuf_v.at[b]
        wv = w_v.at[b]
        for g in range(_K // 16):
            w16 = wv[pl.ds(g * 16, 16)]
            for i in range(16):
                ws = w16[i]
                e = g * 16 + i
                for q in range(_H // 32):
                    bb = rv[e, pl.ds(32 * q, 32)]
                    lo, hi = plsc.unpack(bb, format=plsc.PackFormat.INTERLEAVED)
                    sb[e, pl.ds(32 * q, 16)] = lo * ws
                    sb[e, pl.ds(32 * q + 16, 16)] = hi * ws

    def chunk_body(c, b, first, guard_prefetch):
        for cp in gather_chunk(c, b):
            cp.wait()
        if guard_prefetch:
            @pl.when(c + 1 < _NCHUNK)
            def _():
                for cp in gather_chunk(c + 1, 1 - b):
                    cp.start()
        else:
            for cp in gather_chunk(c + 1, 1 - b):
                cp.start()
        if not first:
            scatter_chunk(c - 1, 1 - b).wait()
        scale_chunk(b)
        scatter_chunk(c, b).start(add=True)

    for cp in gather_chunk(0, 0):
        cp.start()
    chunk_body(0, 0, True, False)

    def outer_body(t, carry):
        chunk_body(2 * t + 1, 1, False, False)
        chunk_body(2 * t + 2, 0, False, True)
        return carry

    lax.fori_loop(0, (_NCHUNK - 1) // 2, outer_body, 0)
    scatter_chunk(_NCHUNK - 1, 0).wait()
    plsc.subcore_barrier()
    pltpu.sync_copy(acc.at[pl.ds(sid * _RPT, _RPT)],
                    out_hbm.at[cid, pl.ds(sid * _RPT, _RPT)])

    @pl.when(sid == _NS - 1)
    def _():
        pltpu.sync_copy(acc.at[pl.ds(_RPT * _NS, _REM)],
                        out_hbm.at[cid, pl.ds(_RPT * _NS, _REM)])


_GPW = 2 * 2048 // _NW


@functools.partial(
    pl.kernel,
    mesh=_mesh,
    out_type=jax.ShapeDtypeStruct((2 * 2048, _H), jnp.float32),
    scratch_types=[
        pltpu.VMEM((_GPW,), jnp.int32),
        pltpu.VMEM((_GPW, _H), jnp.float32),
        pltpu.SemaphoreType.DMA,
    ],
    compiler_params=pltpu.CompilerParams(
        use_tc_tiling_on_sc=False, needs_layout_passes=False),
)
def _gather_sc(table_hbm, idx_hbm, out_hbm, idx_v, rows_v, sem):
    cid = lax.axis_index("c")
    sid = lax.axis_index("s")
    wid = sid * _NC + cid
    base = wid * _GPW
    pltpu.sync_copy(idx_hbm.at[pl.ds(base, _GPW)], idx_v)
    pltpu.async_copy(table_hbm.at[idx_v], rows_v, sem).wait()
    pltpu.sync_copy(rows_v, out_hbm.at[pl.ds(base, _GPW)])


_BLK = 1000


def _mm0_body(x_ref, w_ref, b_ref, o_ref):
    o_ref[...] = (
        jnp.dot(x_ref[...], w_ref[...], preferred_element_type=jnp.float32)
        + b_ref[...]
    ).astype(jnp.bfloat16)


def _mm0(x, W, b2):
    return pl.pallas_call(
        _mm0_body,
        grid=(_N // _BLK,),
        in_specs=[
            pl.BlockSpec((_BLK, _D), lambda i: (i, 0)),
            pl.BlockSpec((_D, _H), lambda i: (0, 0)),
            pl.BlockSpec((1, _H), lambda i: (0, 0)),
        ],
        out_specs=pl.BlockSpec((_BLK, _H), lambda i: (i, 0)),
        out_shape=jax.ShapeDtypeStruct((_N, _H), jnp.bfloat16),
    )(x, W, b2)


def _fuse_body(p_ref, s_ref, t_ref, w_ref, o_ref):
    x = p_ref[0] + p_ref[1]
    y = jnp.maximum(x * s_ref[...] + t_ref[...], 0.0)
    o_ref[...] = jnp.dot(
        y, w_ref[...], preferred_element_type=jnp.float32
    ).astype(jnp.bfloat16)


def _fuse_mm(p, scale2, shift2, W):
    out_cols = W.shape[1]
    return pl.pallas_call(
        _fuse_body,
        grid=(_N // _BLK,),
        in_specs=[
            pl.BlockSpec((_NC, _BLK, _H), lambda i: (0, i, 0)),
            pl.BlockSpec((1, _H), lambda i: (0, 0)),
            pl.BlockSpec((1, _H), lambda i: (0, 0)),
            pl.BlockSpec((_H, out_cols), lambda i: (0, 0)),
        ],
        out_specs=pl.BlockSpec((_BLK, out_cols), lambda i: (i, 0)),
        out_shape=jax.ShapeDtypeStruct((_N, out_cols), jnp.bfloat16),
    )(p, scale2, shift2, W)


def _fuse_mm_bias_body(p_ref, s_ref, t_ref, w_ref, b_ref, o_ref):
    x = p_ref[0] + p_ref[1]
    y = jnp.maximum(x * s_ref[...] + t_ref[...], 0.0)
    o_ref[...] = (
        jnp.dot(y, w_ref[...], preferred_element_type=jnp.float32) + b_ref[...]
    )


def _fuse_mm_bias(p, scale2, shift2, W, b2, blk):
    rows = p.shape[1]
    out_cols = W.shape[1]
    return pl.pallas_call(
        _fuse_mm_bias_body,
        grid=(rows // blk,),
        in_specs=[
            pl.BlockSpec((_NC, blk, _H), lambda i: (0, i, 0)),
            pl.BlockSpec((1, _H), lambda i: (0, 0)),
            pl.BlockSpec((1, _H), lambda i: (0, 0)),
            pl.BlockSpec((_H, out_cols), lambda i: (0, 0)),
            pl.BlockSpec((1, out_cols), lambda i: (0, 0)),
        ],
        out_specs=pl.BlockSpec((blk, out_cols), lambda i: (i, 0)),
        out_shape=jax.ShapeDtypeStruct((rows, out_cols), jnp.float32),
    )(p, scale2, shift2, W, b2)


_PERM_LIST = [0] * _H
for _j in range(_H // 32):
    for _i in range(16):
        _PERM_LIST[32 * _j + 2 * _i] = 32 * _j + _i
        _PERM_LIST[32 * _j + 2 * _i + 1] = 32 * _j + 16 + _i


def kernel(features, edge_index, edge_weight, idx,
           W0, b0, bias0, gamma0, beta0, mean0, var0,
           W1, bias1, gamma1, beta1, mean1, var1,
           Wf, bf):
    src = edge_index[0]
    dst = edge_index[1].reshape(_NW, _NCHUNK, _K)
    zeros = jnp.zeros((_RPT, _H), jnp.float32)
    perm = jnp.array(_PERM_LIST, dtype=jnp.int32)

    scale0 = lax.rsqrt(var0 + _EPS) * gamma0
    shift0 = beta0 + (bias0 - mean0) * scale0
    scale1 = lax.rsqrt(var1 + _EPS) * gamma1
    shift1 = beta1 + (bias1 - mean1) * scale1

    h = _mm0(features, W0[:, perm], b0[perm].reshape(1, _H))
    p = _spmm_sc(src, dst, edge_weight, h, zeros)
    h = _fuse_mm(p, scale0.reshape(1, _H), shift0.reshape(1, _H), W1[:, perm])
    p = _spmm_sc(src, dst, edge_weight, h, zeros)
    idx2 = jnp.concatenate([idx, idx + _N])
    g = _gather_sc(p.reshape(2 * _N, _H), idx2)
    g = g.reshape(_NC, 2048, _H)
    return _fuse_mm_bias(g, scale1.reshape(1, _H), shift1.reshape(1, _H),
                         Wf, bf.reshape(1, _C), 1024)


# speedup vs baseline: 2.6070x; 1.0396x over previous
import functools

import jax
import jax.numpy as jnp
from jax import lax
from jax.experimental import pallas as pl
from jax.experimental.pallas import tpu as pltpu
from jax.experimental.pallas import tpu_sc as plsc

_N = 10000
_E = 320000
_D = 128
_H = 128
_C = 16
_EPS = 1e-05

_NC = 2
_NS = 16
_NW = _NC * _NS
_EPW = _E // _NW
_K = 80
_NCHUNK = _EPW // _K
_RPT = 624
_REM = _N - _RPT * _NS

_mesh = plsc.VectorSubcoreMesh(core_axis_name="c", subcore_axis_name="s")


_GPW = 2048 // _NS


def _spmm_body(final, src_hbm, dst3_hbm, w_hbm, h_hbm, zeros_hbm, idx_hbm,
               out_hbm, src_v, dst_v, w_v, rows_v, sbuf_v, acc, gsem, ssem,
               idx_v):
    cid = lax.axis_index("c")
    sid = lax.axis_index("s")
    wid = sid * _NC + cid

    pltpu.sync_copy(zeros_hbm, acc.at[pl.ds(sid * _RPT, _RPT)])

    @pl.when(sid == _NS - 1)
    def _():
        pltpu.sync_copy(zeros_hbm.at[pl.ds(0, _REM)],
                        acc.at[pl.ds(_RPT * _NS, _REM)])

    pltpu.sync_copy(src_hbm.at[pl.ds(wid * _EPW, _EPW)], src_v)
    plsc.subcore_barrier()

    def gather_chunk(c, buf):
        return (
            pltpu.make_async_copy(
                h_hbm.at[src_v.at[pl.ds(c * _K, _K)]], rows_v.at[buf], gsem),
            pltpu.make_async_copy(
                w_hbm.at[pl.ds(wid * _EPW + c * _K, _K)], w_v.at[buf], gsem),
            pltpu.make_async_copy(
                dst3_hbm.at[wid, c], dst_v.at[lax.rem(c, 3)], gsem),
        )

    def scatter_chunk(c, buf):
        return pltpu.make_async_copy(
            sbuf_v.at[buf], acc.at[dst_v.at[lax.rem(c, 3)]], ssem)

    def scale_chunk(b):
        rv = rows_v.at[b]
        sb = sbuf_v.at[b]
        wv = w_v.at[b]
        for g in range(_K // 16):
            w16 = wv[pl.ds(g * 16, 16)]
            for i in range(16):
                ws = w16[i]
                e = g * 16 + i
                for q in range(_H // 32):
                    bb = rv[e, pl.ds(32 * q, 32)]
                    lo, hi = plsc.unpack(bb, format=plsc.PackFormat.INTERLEAVED)
                    sb[e, pl.ds(32 * q, 16)] = lo * ws
                    sb[e, pl.ds(32 * q + 16, 16)] = hi * ws

    def chunk_body(c, b, first, guard_prefetch):
        for cp in gather_chunk(c, b):
            cp.wait()
        if guard_prefetch:
            @pl.when(c + 1 < _NCHUNK)
            def _():
                for cp in gather_chunk(c + 1, 1 - b):
                    cp.start()
        else:
            for cp in gather_chunk(c + 1, 1 - b):
                cp.start()
        if not first:
            scatter_chunk(c - 1, 1 - b).wait()
        scale_chunk(b)
        scatter_chunk(c, b).start(add=True)

    for cp in gather_chunk(0, 0):
        cp.start()
    chunk_body(0, 0, True, False)

    def outer_body(t, carry):
        chunk_body(2 * t + 1, 1, False, False)
        chunk_body(2 * t + 2, 0, False, True)
        return carry

    lax.fori_loop(0, (_NCHUNK - 1) // 2, outer_body, 0)
    scatter_chunk(_NCHUNK - 1, 0).wait()
    plsc.subcore_barrier()
    if not final:
        pltpu.sync_copy(acc.at[pl.ds(sid * _RPT, _RPT)],
                        out_hbm.at[cid, pl.ds(sid * _RPT, _RPT)])

        @pl.when(sid == _NS - 1)
        def _():
            pltpu.sync_copy(acc.at[pl.ds(_RPT * _NS, _REM)],
                            out_hbm.at[cid, pl.ds(_RPT * _NS, _REM)])
    else:
        pltpu.sync_copy(idx_hbm.at[pl.ds(sid * _GPW, _GPW)], idx_v)
        g0 = pltpu.make_async_copy(
            acc.at[idx_v.at[pl.ds(0, 64)]], sbuf_v.at[0].at[pl.ds(0, 64)],
            gsem)
        g1 = pltpu.make_async_copy(
            acc.at[idx_v.at[pl.ds(64, 64)]], sbuf_v.at[1].at[pl.ds(0, 64)],
            gsem)
        g0.start()
        g1.start()
        g0.wait()
        g1.wait()
        pltpu.sync_copy(sbuf_v.at[0].at[pl.ds(0, 64)],
                        out_hbm.at[cid, pl.ds(sid * _GPW, 64)])
        pltpu.sync_copy(sbuf_v.at[1].at[pl.ds(0, 64)],
                        out_hbm.at[cid, pl.ds(sid * _GPW + 64, 64)])


def _make_spmm(final):
    out_rows = _GPW * _NS if final else _N
    scratch = [
        pltpu.VMEM((_EPW,), jnp.int32),
        pltpu.VMEM((3, _K), jnp.int32),
        pltpu.VMEM((2, _K), jnp.float32),
        pltpu.VMEM((2, _K, _H), jnp.bfloat16),
        pltpu.VMEM((2, _K, _H), jnp.float32),
        pltpu.VMEM_SHARED((_N, _H), jnp.float32),
        pltpu.SemaphoreType.DMA,
        pltpu.SemaphoreType.DMA,
        pltpu.VMEM((_GPW,), jnp.int32),
    ]
    kern = functools.partial(
        pl.kernel,
        mesh=_mesh,
        out_type=jax.ShapeDtypeStruct((_NC, out_rows, _H), jnp.float32),
        scratch_types=scratch,
        compiler_params=pltpu.CompilerParams(
            use_tc_tiling_on_sc=False, needs_layout_passes=False),
    )
    if final:
        @kern
        def spmm(src_hbm, dst3_hbm, w_hbm, h_hbm, zeros_hbm, idx_hbm,
                 out_hbm, *rest):
            _spmm_body(True, src_hbm, dst3_hbm, w_hbm, h_hbm, zeros_hbm,
                       idx_hbm, out_hbm, *rest)
    else:
        @kern
        def spmm(src_hbm, dst3_hbm, w_hbm, h_hbm, zeros_hbm, out_hbm, *rest):
            _spmm_body(False, src_hbm, dst3_hbm, w_hbm, h_hbm, zeros_hbm,
                       None, out_hbm, *rest)
    return spmm


_spmm_mid = _make_spmm(False)
_spmm_final = _make_spmm(True)


_BLK = 1000


def _mm0_body(x_ref, w_ref, b_ref, o_ref):
    o_ref[...] = (
        jnp.dot(x_ref[...], w_ref[...], preferred_element_type=jnp.float32)
        + b_ref[...]
    ).astype(jnp.bfloat16)


def _mm0(x, W, b2):
    return pl.pallas_call(
        _mm0_body,
        grid=(_N // _BLK,),
        in_specs=[
            pl.BlockSpec((_BLK, _D), lambda i: (i, 0)),
            pl.BlockSpec((_D, _H), lambda i: (0, 0)),
            pl.BlockSpec((1, _H), lambda i: (0, 0)),
        ],
        out_specs=pl.BlockSpec((_BLK, _H), lambda i: (i, 0)),
        out_shape=jax.ShapeDtypeStruct((_N, _H), jnp.bfloat16),
    )(x, W, b2)


def _fuse_body(p_ref, s_ref, t_ref, w_ref, o_ref):
    x = p_ref[0] + p_ref[1]
    y = jnp.maximum(x * s_ref[...] + t_ref[...], 0.0)
    o_ref[...] = jnp.dot(
        y, w_ref[...], preferred_element_type=jnp.float32
    ).astype(jnp.bfloat16)


def _fuse_mm(p, scale2, shift2, W):
    out_cols = W.shape[1]
    return pl.pallas_call(
        _fuse_body,
        grid=(_N // _BLK,),
        in_specs=[
            pl.BlockSpec((_NC, _BLK, _H), lambda i: (0, i, 0)),
            pl.BlockSpec((1, _H), lambda i: (0, 0)),
            pl.BlockSpec((1, _H), lambda i: (0, 0)),
            pl.BlockSpec((_H, out_cols), lambda i: (0, 0)),
        ],
        out_specs=pl.BlockSpec((_BLK, out_cols), lambda i: (i, 0)),
        out_shape=jax.ShapeDtypeStruct((_N, out_cols), jnp.bfloat16),
    )(p, scale2, shift2, W)


def _fuse_mm_bias_body(p_ref, s_ref, t_ref, w_ref, b_ref, o_ref):
    x = p_ref[0] + p_ref[1]
    y = jnp.maximum(x * s_ref[...] + t_ref[...], 0.0)
    o_ref[...] = (
        jnp.dot(y, w_ref[...], preferred_element_type=jnp.float32) + b_ref[...]
    )


def _fuse_mm_bias(p, scale2, shift2, W, b2, blk):
    rows = p.shape[1]
    out_cols = W.shape[1]
    return pl.pallas_call(
        _fuse_mm_bias_body,
        grid=(rows // blk,),
        in_specs=[
            pl.BlockSpec((_NC, blk, _H), lambda i: (0, i, 0)),
            pl.BlockSpec((1, _H), lambda i: (0, 0)),
            pl.BlockSpec((1, _H), lambda i: (0, 0)),
            pl.BlockSpec((_H, out_cols), lambda i: (0, 0)),
            pl.BlockSpec((1, out_cols), lambda i: (0, 0)),
        ],
        out_specs=pl.BlockSpec((blk, out_cols), lambda i: (i, 0)),
        out_shape=jax.ShapeDtypeStruct((rows, out_cols), jnp.float32),
    )(p, scale2, shift2, W, b2)


_PERM_LIST = [0] * _H
for _j in range(_H // 32):
    for _i in range(16):
        _PERM_LIST[32 * _j + 2 * _i] = 32 * _j + _i
        _PERM_LIST[32 * _j + 2 * _i + 1] = 32 * _j + 16 + _i


def kernel(features, edge_index, edge_weight, idx,
           W0, b0, bias0, gamma0, beta0, mean0, var0,
           W1, bias1, gamma1, beta1, mean1, var1,
           Wf, bf):
    src = edge_index[0]
    dst = edge_index[1].reshape(_NW, _NCHUNK, _K)
    zeros = jnp.zeros((_RPT, _H), jnp.float32)
    perm = jnp.array(_PERM_LIST, dtype=jnp.int32)

    scale0 = lax.rsqrt(var0 + _EPS) * gamma0
    shift0 = beta0 + (bias0 - mean0) * scale0
    scale1 = lax.rsqrt(var1 + _EPS) * gamma1
    shift1 = beta1 + (bias1 - mean1) * scale1

    h = _mm0(features, W0[:, perm], b0[perm].reshape(1, _H))
    p = _spmm_mid(src, dst, edge_weight, h, zeros)
    h = _fuse_mm(p, scale0.reshape(1, _H), shift0.reshape(1, _H), W1[:, perm])
    g = _spmm_final(src, dst, edge_weight, h, zeros, idx)
    return _fuse_mm_bias(g, scale1.reshape(1, _H), shift1.reshape(1, _H),
                         Wf, bf.reshape(1, _C), 1024)
